# baseline (speedup 1.0000x reference)
import functools
import jax
import jax.numpy as jnp
from jax import lax
from jax.experimental import pallas as pl
from jax.experimental.pallas import tpu as pltpu
from jax.experimental.pallas import tpu_sc as plsc

NUM_FRAMES = 4096
NUM_CLASSES = 1024
NC = 2
NS = 16
L = 16
NW = NC * NS
ROWS_PER_W = NUM_FRAMES // NW
CHUNK = 32
NCHUNK = ROWS_PER_W // CHUNK
SUB = CHUNK // L
BLANK = 0


def _argmax_16rows(buf, row_off, lane):
    rowbase = (lane + row_off) * NUM_CLASSES

    def step(c, carry):
        maxv, maxj = carry
        v = plsc.load_gather(buf, [rowbase + c])
        m = v > maxv
        return jnp.where(m, v, maxv), jnp.where(m, c, maxj)

    init = (jnp.full((L,), -jnp.inf, jnp.float32), jnp.zeros((L,), jnp.int32))
    _, maxj = lax.fori_loop(0, NUM_CLASSES, step, init, unroll=4)
    return maxj


def _body(em_hbm, out_hbm, buf0, buf1, bufp, labext, decv, sem0, sem1, semp):
    wid = lax.axis_index("s") * NC + lax.axis_index("c")
    row0 = wid * ROWS_PER_W
    lane = lax.broadcasted_iota(jnp.int32, (L,), 0)

    rp = jnp.maximum(row0 - 1, 0)
    cpp = pltpu.async_copy(
        em_hbm.at[pl.ds(rp * NUM_CLASSES, NUM_CLASSES)], bufp, semp)
    bufs = (buf0, buf1)
    sems = (sem0, sem1)
    cps = [
        pltpu.async_copy(
            em_hbm.at[pl.ds(row0 * NUM_CLASSES, CHUNK * NUM_CLASSES)],
            buf0, sem0),
        None,
    ]

    cpp.wait()

    def pstep(j, carry):
        maxv, maxj = carry
        v = plsc.load_gather(bufp, [lane + j * L])
        m = v > maxv
        return jnp.where(m, v, maxv), jnp.where(m, j, maxj)

    pinit = (jnp.full((L,), -jnp.inf, jnp.float32), jnp.zeros((L,), jnp.int32))
    pmaxv, pmaxj = lax.fori_loop(0, NUM_CLASSES // L, pstep, pinit, unroll=4)
    gmax = lax.reduce_max(pmaxv, (0,))
    pcls = jnp.where(pmaxv == gmax, pmaxj * L + lane, jnp.int32(2**31 - 1))
    plabel = lax.reduce_min(pcls, (0,))
    plabel = jnp.where(wid == 0, jnp.int32(-1), plabel)
    plsc.store_scatter(
        labext, [jnp.zeros((L,), jnp.int32)],
        jnp.zeros((L,), jnp.int32) + plabel, mask=lane == 0)

    for k in range(NCHUNK):
        if k + 1 < NCHUNK:
            cps[(k + 1) % 2] = pltpu.async_copy(
                em_hbm.at[pl.ds((row0 + (k + 1) * CHUNK) * NUM_CLASSES,
                                CHUNK * NUM_CLASSES)],
                bufs[(k + 1) % 2], sems[(k + 1) % 2])
        cps[k % 2].wait()
        for s in range(SUB):
            labels = _argmax_16rows(bufs[k % 2], s * L, lane)
            plsc.store_scatter(
                labext, [lane + (k * CHUNK + s * L + 1)], labels)

    for g in range(ROWS_PER_W // L):
        cur = plsc.load_gather(labext, [lane + g * L + 1])
        prv = plsc.load_gather(labext, [lane + g * L])
        keep = (cur != prv) & (cur != BLANK)
        decv[pl.ds(g * L, L)] = jnp.where(keep, cur, jnp.int32(-1))

    pltpu.sync_copy(decv, out_hbm.at[pl.ds(row0, ROWS_PER_W)])


def kernel(emission):
    mesh = plsc.VectorSubcoreMesh(core_axis_name="c", subcore_axis_name="s")
    k = functools.partial(
        pl.kernel,
        out_type=jax.ShapeDtypeStruct((NUM_FRAMES,), jnp.int32),
        mesh=mesh,
        compiler_params=pltpu.CompilerParams(needs_layout_passes=False),
        scratch_types=[
            pltpu.VMEM((CHUNK * NUM_CLASSES,), jnp.float32),
            pltpu.VMEM((CHUNK * NUM_CLASSES,), jnp.float32),
            pltpu.VMEM((NUM_CLASSES,), jnp.float32),
            pltpu.VMEM((ROWS_PER_W + 1,), jnp.int32),
            pltpu.VMEM((ROWS_PER_W,), jnp.int32),
            pltpu.SemaphoreType.DMA,
            pltpu.SemaphoreType.DMA,
            pltpu.SemaphoreType.DMA,
        ],
    )(_body)
    return k(emission.reshape(NUM_FRAMES * NUM_CLASSES))


# speedup vs baseline: 1.9454x; 1.9454x over previous
import functools
import jax
import jax.numpy as jnp
from jax import lax
from jax.experimental import pallas as pl
from jax.experimental.pallas import tpu as pltpu
from jax.experimental.pallas import tpu_sc as plsc

NUM_FRAMES = 4096
NUM_CLASSES = 1024
NC = 2
NS = 16
L = 16
NW = NC * NS
ROWS_PER_W = NUM_FRAMES // NW
CHUNK = 32
NCHUNK = ROWS_PER_W // CHUNK
NSLICE = NUM_CLASSES // L
NCHAIN = 4
BLANK = 0
BIG = 2**31 - 1


def _row_label(buf, row, lane):
    ninf = jnp.full((L,), -jnp.inf, jnp.float32)
    zero = jnp.zeros((L,), jnp.int32)

    def step(t, carry):
        vs = list(carry[:NCHAIN])
        ss = list(carry[NCHAIN:])
        for k in range(NCHAIN):
            s = t * NCHAIN + k
            v = buf[row, pl.ds(s * L, L)]
            m = v > vs[k]
            vs[k] = jnp.where(m, v, vs[k])
            ss[k] = jnp.where(m, s, ss[k])
        return tuple(vs) + tuple(ss)

    init = (ninf,) * NCHAIN + (zero,) * NCHAIN
    res = lax.fori_loop(0, NSLICE // NCHAIN, step, init)
    vs = list(res[:NCHAIN])
    cs = [res[NCHAIN + k] * L + lane for k in range(NCHAIN)]
    while len(vs) > 1:
        nv, ncl = [], []
        for a in range(0, len(vs), 2):
            b = a + 1
            take_b = (vs[b] > vs[a]) | ((vs[b] == vs[a]) & (cs[b] < cs[a]))
            nv.append(jnp.where(take_b, vs[b], vs[a]))
            ncl.append(jnp.where(take_b, cs[b], cs[a]))
        vs, cs = nv, ncl
    gmax = lax.reduce_max(vs[0], (0,))
    cand = jnp.where(vs[0] == gmax, cs[0], jnp.int32(BIG))
    return lax.reduce_min(cand, (0,))


def _body(em_hbm, out_hbm, buf0, buf1, bufp, labext, decv, sem0, sem1, semp):
    wid = lax.axis_index("s") * NC + lax.axis_index("c")
    row0 = wid * ROWS_PER_W
    lane = lax.broadcasted_iota(jnp.int32, (L,), 0)

    rp = jnp.maximum(row0 - 1, 0)
    cpp = pltpu.async_copy(em_hbm.at[pl.ds(rp, 1), :], bufp, semp)
    bufs = (buf0, buf1)
    sems = (sem0, sem1)
    cps = [pltpu.async_copy(em_hbm.at[pl.ds(row0, CHUNK), :], buf0, sem0),
           None]

    cpp.wait()
    plabel = jnp.where(wid == 0, jnp.int32(-1), _row_label(bufp, 0, lane))
    plsc.store_scatter(
        labext, [jnp.zeros((L,), jnp.int32)],
        jnp.zeros((L,), jnp.int32) + plabel, mask=lane == 0)

    for k in range(NCHUNK):
        if k + 1 < NCHUNK:
            cps[(k + 1) % 2] = pltpu.async_copy(
                em_hbm.at[pl.ds(row0 + (k + 1) * CHUNK, CHUNK), :],
                bufs[(k + 1) % 2], sems[(k + 1) % 2])
        cps[k % 2].wait()
        labs = [_row_label(bufs[k % 2], r, lane) for r in range(CHUNK)]
        zero = jnp.zeros((L,), jnp.int32)
        for r0 in range(0, CHUNK, L):
            lab_vec = zero
            for r in range(L):
                lab_vec = jnp.where(lane == r, labs[r0 + r], lab_vec)
            plsc.store_scatter(labext, [lane + (k * CHUNK + r0 + 1)], lab_vec)

    for g in range(ROWS_PER_W // L):
        cur = plsc.load_gather(labext, [lane + g * L + 1])
        prv = plsc.load_gather(labext, [lane + g * L])
        keep = (cur != prv) & (cur != BLANK)
        decv[pl.ds(g * L, L)] = jnp.where(keep, cur, jnp.int32(-1))

    pltpu.sync_copy(decv, out_hbm.at[pl.ds(row0, ROWS_PER_W)])


def kernel(emission):
    mesh = plsc.VectorSubcoreMesh(core_axis_name="c", subcore_axis_name="s")
    k = functools.partial(
        pl.kernel,
        out_type=jax.ShapeDtypeStruct((NUM_FRAMES,), jnp.int32),
        mesh=mesh,
        compiler_params=pltpu.CompilerParams(needs_layout_passes=False),
        scratch_types=[
            pltpu.VMEM((CHUNK, NUM_CLASSES), jnp.float32),
            pltpu.VMEM((CHUNK, NUM_CLASSES), jnp.float32),
            pltpu.VMEM((1, NUM_CLASSES), jnp.float32),
            pltpu.VMEM((ROWS_PER_W + 1,), jnp.int32),
            pltpu.VMEM((ROWS_PER_W,), jnp.int32),
            pltpu.SemaphoreType.DMA,
            pltpu.SemaphoreType.DMA,
            pltpu.SemaphoreType.DMA,
        ],
    )(_body)
    return k(emission)


# speedup vs baseline: 7.9602x; 4.0919x over previous
import jax
import jax.numpy as jnp
from jax.experimental import pallas as pl
from jax.experimental.pallas import tpu as pltpu

NUM_FRAMES = 4096
NUM_CLASSES = 1024
BLOCK_ROWS = 512
NUM_BLOCKS = NUM_FRAMES // BLOCK_ROWS
BLANK = 0
NEG = -2147483648


def _decode_block(x_ref, out_ref, carry_ref):
    i = pl.program_id(0)

    @pl.when(i == 0)
    def _init():
        carry_ref[0] = jnp.int32(-1)

    x = x_ref[...]
    m = jnp.max(x, axis=1, keepdims=True)
    cls = jax.lax.broadcasted_iota(jnp.int32, x.shape, 1)
    idx = jnp.min(jnp.where(x == m, cls, NUM_CLASSES), axis=1)
    idx = idx.reshape(1, BLOCK_ROWS)

    carry = carry_ref[0]
    pos = jax.lax.broadcasted_iota(jnp.int32, (1, BLOCK_ROWS), 1)
    prev = jnp.where(pos == 0, carry, jnp.roll(idx, 1, axis=1))
    keep = (idx != prev) & (idx != BLANK)
    out_ref[...] = jnp.where(keep, idx, jnp.int32(-1)).reshape(1, 1, BLOCK_ROWS)

    carry_ref[0] = jnp.max(jnp.where(pos == BLOCK_ROWS - 1, idx, NEG))


def kernel(emission):
    out = pl.pallas_call(
        _decode_block,
        grid=(NUM_BLOCKS,),
        in_specs=[
            pl.BlockSpec((BLOCK_ROWS, NUM_CLASSES), lambda i: (i, 0)),
        ],
        out_specs=pl.BlockSpec((1, 1, BLOCK_ROWS), lambda i: (i, 0, 0)),
        out_shape=jax.ShapeDtypeStruct((NUM_BLOCKS, 1, BLOCK_ROWS), jnp.int32),
        scratch_shapes=[pltpu.SMEM((1,), jnp.int32)],
    )(emission)
    return out.reshape(NUM_FRAMES)
